# baseline (speedup 1.0000x reference)
import jax
import jax.numpy as jnp
from jax.experimental import pallas as pl
from jax.experimental.pallas import tpu as pltpu

_SB = 1024
_PAIR = 128


def _f32dot(a, b, ta=False):
    dims = (((0,) if ta else (1,), (0,)), ((), ()))
    return jax.lax.dot_general(a, b, dims, preferred_element_type=jnp.float32)


def _bf16dot(a, b):
    return jax.lax.dot_general(a, b, (((1,), (0,)), ((), ())),
                               preferred_element_type=jnp.float32)


def _affinity(t, vcat_ref, psq_ref, npair):
    parts = []
    for c in range(npair):
        sl = slice(c * _PAIR, (c + 1) * _PAIR)
        tc = t[:, sl]
        g = _f32dot(jnp.concatenate([tc, tc * tc], axis=1), vcat_ref[c])
        parts.append(g + psq_ref[0, :, sl])
    e = jnp.concatenate(parts, axis=1)
    return jnp.exp(jnp.minimum(e, 0.0))


def _body(x_ref, wq_ref, wk_ref, wv_ref, vcat_ref, psq_ref, ma_ref, wo_ref,
          out_ref, wbuf, ss_s, norm_s):
    ph = pl.program_id(1)
    c = pl.program_id(2)
    nchunk = pl.num_programs(2)
    npair = ss_s.shape[0] // _PAIR

    @pl.when(ph == 0)
    def _phase0():
        xb = x_ref[0]
        k = _f32dot(xb, wk_ref[...])
        v = _f32dot(xb, wv_ref[...]).astype(jnp.bfloat16)
        q = _f32dot(xb, wq_ref[...])
        ak = _affinity(k, vcat_ref, psq_ref, npair)
        wbuf[pl.ds(c * _SB, _SB), :] = (
            _affinity(q, vcat_ref, psq_ref, npair).astype(jnp.bfloat16))
        nc = _f32dot(jnp.ones((8, ak.shape[0]), jnp.float32), ak)

        @pl.when(c == 0)
        def _():
            norm_s[...] = nc

        @pl.when(c != 0)
        def _():
            norm_s[...] += nc

        akb = ak.astype(jnp.bfloat16)
        for p in range(npair):
            sl = slice(p * _PAIR, (p + 1) * _PAIR)
            blk = _f32dot(akb[:, sl], v[:, sl], ta=True)

            @pl.when(c == 0)
            def _():
                ss_s[sl, :_PAIR] = blk

            @pl.when(c != 0)
            def _():
                ss_s[sl, :_PAIR] += blk

        @pl.when(c == nchunk - 1)
        def _():
            ncol = jnp.transpose(norm_s[0:1, :])
            for p in range(npair):
                sl = slice(p * _PAIR, (p + 1) * _PAIR)
                ma = ma_ref[sl, :]
                ss_s[sl, :_PAIR] = ss_s[sl, :_PAIR] * ma
                ss_s[sl, _PAIR:] = ma * ncol[sl, :]

    @pl.when(ph == 1)
    def _phase1():
        w = wbuf[pl.ds(c * _SB, _SB), :]
        ys = []
        for p in range(npair):
            sl = slice(p * _PAIR, (p + 1) * _PAIR)
            nd = _bf16dot(w[:, sl], ss_s[sl, :].astype(jnp.bfloat16))
            ys.append(nd[:, :_PAIR] / (nd[:, _PAIR:] + 1e-8))
        y = jnp.concatenate(ys, axis=1).astype(jnp.bfloat16)
        out_ref[0] = _bf16dot(y, wo_ref[...].astype(jnp.bfloat16))


def kernel(x, Wq, Wk, Wv, Wo, positions, log_scales, amplitudes):
    B, S, D = x.shape
    H, K, DH = positions.shape
    f32 = jnp.float32
    npair = H // 2

    scales = jnp.exp(log_scales)
    itv = (0.5 / (scales * scales + 1e-6)).reshape(H * K)
    psq = jnp.sum(positions * positions, axis=-1).reshape(H * K)
    psq_s = (-psq * itv).reshape(1, 1, H * K)
    amp = amplitudes.reshape(H * K)
    eye_h = jnp.eye(H, dtype=f32)
    pbd = jnp.einsum('hg,hkd->hdgk', eye_h, positions).reshape(D, D)
    m2 = jnp.kron(jnp.eye(2, dtype=f32), jnp.ones((DH, K), f32))
    pbd_blocks = jnp.stack([pbd[c * _PAIR:(c + 1) * _PAIR,
                                c * _PAIR:(c + 1) * _PAIR]
                            for c in range(npair)])
    vcat = jnp.concatenate([-2.0 * pbd_blocks,
                            jnp.broadcast_to(m2, (npair, _PAIR, _PAIR))],
                           axis=1) * (-itv.reshape(npair, 1, _PAIR))
    ma = jnp.tile(m2, (npair, 1)) * amp[:, None]

    nchunk = S // _SB
    grid = (B, 2, nchunk)

    full = lambda b, ph, c: (0, 0)
    full3 = lambda b, ph, c: (0, 0, 0)
    xspec = pl.BlockSpec(
        (1, _SB, D),
        lambda b, ph, c: (b, jnp.where(ph == 0, c, nchunk - 1), 0))
    ospec = pl.BlockSpec(
        (1, _SB, D),
        lambda b, ph, c: (b, jnp.where(ph == 0, 0, c), 0))
    wspec = pl.BlockSpec((D, D), full)
    vcspec = pl.BlockSpec((npair, 2 * _PAIR, _PAIR), full3)
    rowspec = pl.BlockSpec((1, 1, D), full3)
    maspec = pl.BlockSpec((D, _PAIR), full)

    out = pl.pallas_call(
        _body,
        grid=grid,
        in_specs=[xspec, wspec, wspec, wspec, vcspec, rowspec, maspec, wspec],
        out_specs=ospec,
        out_shape=jax.ShapeDtypeStruct((B, S, D), f32),
        scratch_shapes=[
            pltpu.VMEM((S, D), jnp.bfloat16),
            pltpu.VMEM((D, 2 * _PAIR), f32),
            pltpu.VMEM((8, D), f32),
        ],
        compiler_params=pltpu.CompilerParams(
            dimension_semantics=("arbitrary", "arbitrary", "arbitrary")),
    )(x, Wq, Wk, Wv, vcat, psq_s, ma, Wo)
    return out


# speedup vs baseline: 1.0297x; 1.0297x over previous
import jax
import jax.numpy as jnp
from jax.experimental import pallas as pl
from jax.experimental.pallas import tpu as pltpu

_SB = 2048
_PAIR = 128


def _f32dot(a, b, ta=False):
    dims = (((0,) if ta else (1,), (0,)), ((), ()))
    return jax.lax.dot_general(a, b, dims, preferred_element_type=jnp.float32)


def _bdot(a, b, ta=False):
    dims = (((0,) if ta else (1,), (0,)), ((), ()))
    return jax.lax.dot_general(a.astype(jnp.bfloat16), b.astype(jnp.bfloat16),
                               dims, preferred_element_type=jnp.float32)


def _affinity(t, vcat_ref, psq_ref, npair):
    parts = []
    for c in range(npair):
        sl = slice(c * _PAIR, (c + 1) * _PAIR)
        tc = t[:, sl]
        g = _f32dot(jnp.concatenate([tc, tc * tc], axis=1), vcat_ref[c])
        parts.append(g + psq_ref[0, :, sl])
    e = jnp.concatenate(parts, axis=1)
    return jnp.exp(jnp.minimum(e, 0.0))


def _pass1_body(x_ref, wk_ref, wv_ref, vcat_ref, psq_ref, ss_ref, norm_ref):
    c = pl.program_id(1)
    npair = ss_ref.shape[1] // _PAIR
    xb = x_ref[0]
    k = _f32dot(xb, wk_ref[...])
    v = _bdot(xb, wv_ref[...])
    ak = _affinity(k, vcat_ref, psq_ref, npair)
    nc = _f32dot(jnp.ones((8, ak.shape[0]), jnp.float32), ak)[0:1]

    akb = ak.astype(jnp.bfloat16)
    vb = v.astype(jnp.bfloat16)
    for p in range(npair):
        sl = slice(p * _PAIR, (p + 1) * _PAIR)
        blk = _f32dot(akb[:, sl], vb[:, sl], ta=True)

        @pl.when(c == 0)
        def _():
            ss_ref[0, sl, :] = blk

        @pl.when(c != 0)
        def _():
            ss_ref[0, sl, :] += blk

    @pl.when(c == 0)
    def _():
        norm_ref[0] = nc

    @pl.when(c != 0)
    def _():
        norm_ref[0] += nc


def _pass2_body(x_ref, wq_ref, vcat_ref, psq_ref, wo_ref, rhs_ref, out_ref):
    npair = rhs_ref.shape[1] // _PAIR
    xb = x_ref[0]
    q = _f32dot(xb, wq_ref[...])
    w = _affinity(q, vcat_ref, psq_ref, npair)
    ys = []
    for p in range(npair):
        sl = slice(p * _PAIR, (p + 1) * _PAIR)
        nd = _bdot(w[:, sl], rhs_ref[0, sl, :])
        ys.append(nd[:, :_PAIR] / (nd[:, _PAIR:] + 1e-8))
    y = jnp.concatenate(ys, axis=1)
    out_ref[0] = _bdot(y, wo_ref[...])


def kernel(x, Wq, Wk, Wv, Wo, positions, log_scales, amplitudes):
    B, S, D = x.shape
    H, K, DH = positions.shape
    f32 = jnp.float32
    npair = H // 2

    scales = jnp.exp(log_scales)
    itv = (0.5 / (scales * scales + 1e-6)).reshape(H * K)
    psq = jnp.sum(positions * positions, axis=-1).reshape(H * K)
    psq_s = (-psq * itv).reshape(1, 1, H * K)
    amp = amplitudes.reshape(H * K)
    eye_h = jnp.eye(H, dtype=f32)
    pbd = jnp.einsum('hg,hkd->hdgk', eye_h, positions).reshape(D, D)
    m2 = jnp.kron(jnp.eye(2, dtype=f32), jnp.ones((DH, K), f32))
    pbd_blocks = jnp.stack([pbd[c * _PAIR:(c + 1) * _PAIR,
                                c * _PAIR:(c + 1) * _PAIR]
                            for c in range(npair)])
    vcat = jnp.concatenate([-2.0 * pbd_blocks,
                            jnp.broadcast_to(m2, (npair, _PAIR, _PAIR))],
                           axis=1) * (-itv.reshape(npair, 1, _PAIR))

    nc = S // _SB
    grid = (B, nc)

    full = lambda b, c: (0, 0)
    full3 = lambda b, c: (0, 0, 0)
    xspec = pl.BlockSpec((1, _SB, D), lambda b, c: (b, c, 0))
    wspec = pl.BlockSpec((D, D), full)
    vcspec = pl.BlockSpec((npair, 2 * _PAIR, _PAIR), full3)
    rowspec = pl.BlockSpec((1, 1, D), full3)
    ss_spec = pl.BlockSpec((1, D, _PAIR), lambda b, c: (b, 0, 0))
    nm_spec = pl.BlockSpec((1, 1, D), lambda b, c: (b, 0, 0))

    ss, norm = pl.pallas_call(
        _pass1_body,
        grid=grid,
        in_specs=[xspec, wspec, wspec, vcspec, rowspec],
        out_specs=[ss_spec, nm_spec],
        out_shape=[jax.ShapeDtypeStruct((B, D, _PAIR), f32),
                   jax.ShapeDtypeStruct((B, 1, D), f32)],
        compiler_params=pltpu.CompilerParams(
            dimension_semantics=("arbitrary", "arbitrary")),
    )(x, Wk, Wv, vcat, psq_s)

    mask = jnp.tile(m2, (npair, 1))
    ss_in = ss * (mask * amp[:, None])[None]
    dmat = (amp * norm[:, 0, :])[..., None] * mask[None]
    rhs = jnp.concatenate([ss_in, dmat], axis=2)
    rhs_spec = pl.BlockSpec((1, D, 2 * _PAIR), lambda b, c: (b, 0, 0))

    out = pl.pallas_call(
        _pass2_body,
        grid=grid,
        in_specs=[xspec, wspec, vcspec, rowspec, wspec, rhs_spec],
        out_specs=xspec,
        out_shape=jax.ShapeDtypeStruct((B, S, D), f32),
        compiler_params=pltpu.CompilerParams(
            dimension_semantics=("parallel", "arbitrary")),
    )(x, Wq, vcat, psq_s, Wo, rhs)
    return out
